# baseline (speedup 1.0000x reference)
import functools

import jax
import jax.numpy as jnp
from jax import lax
from jax.experimental import pallas as pl
from jax.experimental.pallas import tpu as pltpu
from jax.experimental.pallas import tpu_sc as plsc

N = 10000
NPAD = 10240
E = 320000
H = 64
H2 = 2 * H
EPS = 1e-5
NT = 16
NCORE = 2
CH = 64
ECH = 160
GSL = 16
NG = ECH // GSL
EPT = ECH * CH
E_PAD = EPT * NT * NCORE
NPT = NPAD // NT
ROWS = 1024

_mesh = plsc.VectorSubcoreMesh(core_axis_name="c", subcore_axis_name="s")


@functools.partial(
    pl.kernel,
    out_type=[
        jax.ShapeDtypeStruct((NCORE * 2 * NPAD,), jnp.float32),
        jax.ShapeDtypeStruct((E_PAD // CH, CH), jnp.float32),
    ],
    mesh=_mesh,
    scratch_types=[
        pltpu.VMEM((ECH, CH), jnp.int32),
        pltpu.VMEM((ECH, CH), jnp.float32),
        pltpu.VMEM((ECH, CH), jnp.int32),
        pltpu.VMEM((ECH, CH), jnp.int32),
        pltpu.VMEM((ECH, CH), jnp.float32),
        pltpu.VMEM((2 * NPAD // NT,), jnp.float32),
        pltpu.VMEM_SHARED((2 * NPAD,), jnp.float32),
    ],
)
def _sc_deg(dst_hbm, ew_hbm, et_hbm, deg_hbm, e0_hbm,
            dstv, ewv, etv, dstadjv, ewm0v, zv, sh_deg):
    c = lax.axis_index("c")
    s = lax.axis_index("s")
    w = c * NT + s
    npt2 = 2 * NPAD // NT
    pltpu.sync_copy(dst_hbm.at[pl.ds(w * ECH, ECH)], dstv)
    pltpu.sync_copy(ew_hbm.at[pl.ds(w * ECH, ECH)], ewv)
    pltpu.sync_copy(et_hbm.at[pl.ds(w * ECH, ECH)], etv)
    for k in range(npt2 // 16):
        zv[pl.ds(k * 16, 16)] = jnp.zeros((16,), jnp.float32)
    pltpu.sync_copy(zv, sh_deg.at[pl.ds(s * npt2, npt2)])

    def mask_body(j, carry):
        for k in range(CH // 16):
            sl = pl.ds(k * 16, 16)
            et16 = etv[j, sl]
            dstadjv[j, sl] = dstv[j, sl] + et16 * NPAD
            ewm0v[j, sl] = jnp.where(et16 == 0, ewv[j, sl],
                                     jnp.zeros((16,), jnp.float32))
        return carry
    lax.fori_loop(0, ECH, mask_body, 0)
    pltpu.sync_copy(ewm0v, e0_hbm.at[pl.ds(w * ECH, ECH)])
    plsc.subcore_barrier()

    def add_body(j, carry):
        pltpu.sync_copy(ewv.at[j], sh_deg.at[dstadjv.at[j]], add=True)
        return carry
    lax.fori_loop(0, ECH, add_body, 0)
    plsc.subcore_barrier()
    pltpu.sync_copy(sh_deg.at[pl.ds(s * npt2, npt2)],
                    deg_hbm.at[pl.ds(c * 2 * NPAD + s * npt2, npt2)])


@functools.partial(
    pl.kernel,
    out_type=jax.ShapeDtypeStruct((NCORE * NPAD, H2), jnp.float32),
    mesh=_mesh,
    scratch_types=[
        pltpu.VMEM((2, GSL, CH), jnp.int32),
        pltpu.VMEM((2, GSL, CH), jnp.int32),
        pltpu.VMEM((2, GSL, CH), jnp.float32),
        pltpu.VMEM((2, GSL, CH), jnp.float32),
        pltpu.VMEM((4, CH, H2), jnp.float32),
        pltpu.VMEM_SHARED((NPAD, H2), jnp.float32),
        pltpu.SemaphoreType.DMA((4,)),
        pltpu.SemaphoreType.DMA((4,)),
        pltpu.SemaphoreType.DMA((2,)),
    ],
)
def _sc_agg(src_hbm, dst_hbm, ew_hbm, e0_hbm, y_hbm, z_hbm,
            src_st, dst_st, ew_st, e0_st, rows, sh_z,
            g_sem, s_sem, st_sem):
    c = lax.axis_index("c")
    s = lax.axis_index("s")
    w = c * NT + s
    tb = w * ECH

    def zrow(r, carry):
        for kk in range(H2 // 16):
            rows[0, r, pl.ds(kk * 16, 16)] = jnp.zeros((16,), jnp.float32)
        return carry
    lax.fori_loop(0, CH, zrow, 0)
    for k in range(NPT // CH):
        pltpu.sync_copy(rows.at[0], sh_z.at[pl.ds(s * NPT + k * CH, CH)])
    plsc.subcore_barrier()

    def issue_stage(g, p):
        b0 = pl.multiple_of(tb + g * GSL, 8)
        pltpu.async_copy(src_hbm.at[pl.ds(b0, GSL)], src_st.at[p],
                         st_sem.at[p])
        pltpu.async_copy(dst_hbm.at[pl.ds(b0, GSL)], dst_st.at[p],
                         st_sem.at[p])
        pltpu.async_copy(ew_hbm.at[pl.ds(b0, GSL)], ew_st.at[p],
                         st_sem.at[p])
        pltpu.async_copy(e0_hbm.at[pl.ds(b0, GSL)], e0_st.at[p],
                         st_sem.at[p])

    def drain_stage(p):
        pltpu.make_async_copy(src_hbm.at[pl.ds(0, GSL)], src_st.at[p],
                              st_sem.at[p]).wait()
        pltpu.make_async_copy(dst_hbm.at[pl.ds(0, GSL)], dst_st.at[p],
                              st_sem.at[p]).wait()
        pltpu.make_async_copy(ew_hbm.at[pl.ds(0, GSL)], ew_st.at[p],
                              st_sem.at[p]).wait()
        pltpu.make_async_copy(e0_hbm.at[pl.ds(0, GSL)], e0_st.at[p],
                              st_sem.at[p]).wait()

    def drain_scatter(bb):
        pltpu.make_async_copy(rows.at[bb], sh_z.at[pl.ds(0, CH)],
                              s_sem.at[bb]).wait()

    issue_stage(0, 0)
    drain_stage(0)
    issue_stage(1, 1)
    pltpu.async_copy(y_hbm.at[src_st.at[0, 0]], rows.at[0], g_sem.at[0])
    pltpu.async_copy(y_hbm.at[src_st.at[0, 1]], rows.at[1], g_sem.at[1])

    def slot(j, carry):
        jm = lax.rem(j, GSL)
        g = lax.div(j, GSL)
        p = lax.rem(g, 2)
        b = lax.rem(j, 4)

        @pl.when(jnp.logical_and(jm == 2,
                                 jnp.logical_and(j >= GSL,
                                                 j < (NG - 1) * GSL)))
        def _():
            issue_stage(g + 1, lax.rem(g + 1, 2))

        @pl.when(jnp.logical_and(jm == GSL - 2, j < (NG - 1) * GSL))
        def _():
            drain_stage(lax.rem(g + 1, 2))

        jj = j + 2
        bb = lax.rem(jj, 4)
        pj = lax.rem(lax.div(jj, GSL), 2)
        jjm = lax.rem(jj, GSL)

        @pl.when(jj >= 4)
        def _():
            drain_scatter(bb)

        @pl.when(jj < ECH)
        def _():
            pltpu.async_copy(y_hbm.at[src_st.at[pj, jjm]], rows.at[bb],
                             g_sem.at[bb])

        pltpu.make_async_copy(y_hbm.at[src_st.at[0, 0]], rows.at[b],
                              g_sem.at[b]).wait()

        def srow(gg, carry2):
            ew16 = ew_st[p, jm, pl.ds(gg * 16, 16)]
            e016 = e0_st[p, jm, pl.ds(gg * 16, 16)]
            for l in range(16):
                r = gg * 16 + l
                sp0 = jnp.full((16,), e016[l], jnp.float32)
                sp1 = jnp.full((16,), ew16[l], jnp.float32) - sp0
                for kk in range(H // 16):
                    rows[b, r, pl.ds(kk * 16, 16)] = (
                        rows[b, r, pl.ds(kk * 16, 16)] * sp0)
                for kk in range(H // 16):
                    rows[b, r, pl.ds(H + kk * 16, 16)] = (
                        rows[b, r, pl.ds(H + kk * 16, 16)] * sp1)
            return carry2
        lax.fori_loop(0, CH // 16, srow, 0)

        pltpu.async_copy(rows.at[b], sh_z.at[dst_st.at[p, jm]],
                         s_sem.at[b], add=True)
        return carry
    lax.fori_loop(0, ECH, slot, 0)

    drain_scatter(2)
    drain_scatter(3)
    plsc.subcore_barrier()
    pltpu.sync_copy(sh_z.at[pl.ds(s * NPT, NPT)],
                    z_hbm.at[pl.ds(c * NPAD + s * NPT, NPT)])


def _tc_call(body, grid, in_specs, out_specs, out_shape):
    return pl.pallas_call(
        body, grid=grid, in_specs=in_specs, out_specs=out_specs,
        out_shape=out_shape)


def _row_spec(d):
    return pl.BlockSpec((ROWS, d), lambda i: (i, 0))


def _row2_spec(d):
    return pl.BlockSpec((NCORE, ROWS, d), lambda i: (0, i, 0))


def _full_spec(shape):
    return pl.BlockSpec(shape, lambda i: (0,) * len(shape))


def _tc_h_body(x_ref, w_ref, b_ref, o_ref):
    o_ref[...] = jax.nn.relu(
        jnp.dot(x_ref[...], w_ref[...], preferred_element_type=jnp.float32)
        + b_ref[...])


def _tc_b_body(h_ref, deg_ref, w0_ref, w1_ref, y_ref, dis_ref):
    deg = deg_ref[0] + deg_ref[1]
    dis = 1.0 / jnp.sqrt(deg + 1.0)
    h = h_ref[...]
    y0 = dis[0][:, None] * jnp.dot(h, w0_ref[...],
                                   preferred_element_type=jnp.float32)
    y1 = dis[1][:, None] * jnp.dot(h, w1_ref[...],
                                   preferred_element_type=jnp.float32)
    y_ref[:, 0:H] = y0
    y_ref[:, H:H2] = y1
    dis_ref[...] = dis


def _tc_c_body(z_ref, y_ref, dis_ref, b_ref, g_ref, be_ref, w_ref,
               o_ref, y2_ref):
    inv = 1.0 / jnp.sqrt(1.0 + EPS)
    zz = z_ref[0] + z_ref[1] + y_ref[...]
    y2 = []
    for t in range(2):
        dis = dis_ref[t][:, None]
        agg = dis * zz[:, t * H:(t + 1) * H] + b_ref[t]
        o = jax.nn.relu(g_ref[t] * (agg * inv) + be_ref[t])
        o_ref[t] = o
        y2t = dis * jnp.dot(o, w_ref[t], preferred_element_type=jnp.float32)
        y2_ref[:, t * H:(t + 1) * H] = y2t


def _tc_d_body(z_ref, y2_ref, o_ref, dis_ref, b_ref, g_ref, be_ref, x_ref):
    inv = 1.0 / jnp.sqrt(1.0 + EPS)
    zz = z_ref[0] + z_ref[1] + y2_ref[...]
    for t in range(2):
        dis = dis_ref[t][:, None]
        agg = dis * zz[:, t * H:(t + 1) * H] + b_ref[t]
        x_ref[t] = g_ref[t] * (agg * inv) + be_ref[t] + o_ref[t]


def _tc_s_body(x_ref, seed_ref, wcd_ref, bp1_ref, sb_ref):
    sd = seed_ref[0]
    x0r = x_ref[0, pl.ds(sd, 1), :]
    x1r = x_ref[1, pl.ds(sd, 1), :]
    sb_ref[...] = (
        jnp.dot(x0r, wcd_ref[0], preferred_element_type=jnp.float32)
        + jnp.dot(x1r, wcd_ref[1], preferred_element_type=jnp.float32)
        + bp1_ref[...])


def _tc_e_body(x_ref, sb_ref, wab_ref, w2_ref, b2_ref, w3_ref, b3_ref, o_ref):
    o = jax.nn.relu(
        jnp.dot(x_ref[0], wab_ref[0], preferred_element_type=jnp.float32)
        + jnp.dot(x_ref[1], wab_ref[1], preferred_element_type=jnp.float32)
        + sb_ref[...])
    o = jax.nn.relu(jnp.dot(o, w2_ref[...], preferred_element_type=jnp.float32)
                    + b2_ref[...])
    o_ref[...] = jnp.dot(o, w3_ref[...], preferred_element_type=jnp.float32) \
        + b3_ref[...]


def kernel(x, edge_index, edge_type, edge_weight, seed_node_id, W_ft, b_ft, W00, b00, g00, be00, W01, b01, g01, be01, W10, b10, g10, be10, W11, b11, g11, be11, Wp1, bp1, Wp2, bp2, Wp3, bp3):
    f32 = jnp.float32
    grid = (NPAD // ROWS,)

    xp = jnp.pad(x, ((0, NPAD - N), (0, 0)))
    nrow = E_PAD // CH
    src = jnp.pad(edge_index[0], (0, E_PAD - E)).reshape(nrow, CH)
    dst = jnp.pad(edge_index[1], (0, E_PAD - E)).reshape(nrow, CH)
    ew = jnp.pad(edge_weight, (0, E_PAD - E)).reshape(nrow, CH)
    et = jnp.pad(edge_type, (0, E_PAD - E)).reshape(nrow, CH)
    seed = jnp.asarray(seed_node_id, jnp.int32).reshape(1)
    bL0 = jnp.stack([b00, b10]); gL0 = jnp.stack([g00, g10])
    beL0 = jnp.stack([be00, be10])
    WL1 = jnp.stack([W01, W11])
    bL1 = jnp.stack([b01, b11]); gL1 = jnp.stack([g01, g11])
    beL1 = jnp.stack([be01, be11])
    Wp1ab = jnp.stack([Wp1[0:H], Wp1[H:2 * H]])
    Wp1cd = jnp.stack([Wp1[2 * H:3 * H], Wp1[3 * H:4 * H]])

    deg, ewm0 = _sc_deg(dst, ew, et)
    deg4 = deg.reshape(NCORE, 2, NPAD)

    h = _tc_call(
        _tc_h_body, grid,
        [_row_spec(128), _full_spec((128, H)), _full_spec((H,))],
        _row_spec(H), jax.ShapeDtypeStruct((NPAD, H), f32))(xp, W_ft, b_ft)

    y, dis = _tc_call(
        _tc_b_body, grid,
        [_row_spec(H), pl.BlockSpec((NCORE, 2, ROWS), lambda i: (0, 0, i)),
         _full_spec((H, H)), _full_spec((H, H))],
        [_row_spec(H2), pl.BlockSpec((2, ROWS), lambda i: (0, i))],
        [jax.ShapeDtypeStruct((NPAD, H2), f32),
         jax.ShapeDtypeStruct((2, NPAD), f32)])(h, deg4, W00, W10)

    z = _sc_agg(src, dst, ew, ewm0, y)
    z = z.reshape(NCORE, NPAD, H2)

    o, y2 = _tc_call(
        _tc_c_body, grid,
        [_row2_spec(H2), _row_spec(H2),
         pl.BlockSpec((2, ROWS), lambda i: (0, i)),
         _full_spec((2, H)), _full_spec((2, H)),
         _full_spec((2, H)), _full_spec((2, H, H))],
        [_row2_spec(H), _row_spec(H2)],
        [jax.ShapeDtypeStruct((NCORE, NPAD, H), f32),
         jax.ShapeDtypeStruct((NPAD, H2), f32)])(
             z, y, dis, bL0, gL0, beL0, WL1)

    z2 = _sc_agg(src, dst, ew, ewm0, y2)
    z2 = z2.reshape(NCORE, NPAD, H2)

    x01 = _tc_call(
        _tc_d_body, grid,
        [_row2_spec(H2), _row_spec(H2), _row2_spec(H),
         pl.BlockSpec((2, ROWS), lambda i: (0, i)),
         _full_spec((2, H)), _full_spec((2, H)), _full_spec((2, H))],
        _row2_spec(H),
        jax.ShapeDtypeStruct((NCORE, NPAD, H), f32))(
            z2, y2, o, dis, bL1, gL1, beL1)

    sb = pl.pallas_call(
        _tc_s_body,
        grid=(1,),
        in_specs=[_full_spec((NCORE, NPAD, H)),
                  pl.BlockSpec(memory_space=pltpu.SMEM),
                  _full_spec((NCORE, H, H)), _full_spec((H,))],
        out_specs=_full_spec((1, H)),
        out_shape=jax.ShapeDtypeStruct((1, H), f32))(x01, seed, Wp1cd, bp1)

    res = _tc_call(
        _tc_e_body, grid,
        [_row2_spec(H), _full_spec((1, H)), _full_spec((NCORE, H, H)),
         _full_spec((H, H)), _full_spec((H,)), _full_spec((H, 1)),
         _full_spec((1,))],
        _row_spec(1),
        jax.ShapeDtypeStruct((NPAD, 1), f32))(
            x01, sb, Wp1ab, Wp2, bp2, Wp3, bp3)

    return res[:N, 0]


# speedup vs baseline: 1.1088x; 1.1088x over previous
import functools

import jax
import jax.numpy as jnp
from jax import lax
from jax.experimental import pallas as pl
from jax.experimental.pallas import tpu as pltpu
from jax.experimental.pallas import tpu_sc as plsc

N = 10000
NPAD = 10240
E = 320000
H = 64
H2 = 2 * H
EPS = 1e-5
NT = 16
NCORE = 2
CH = 64
ECH = 160
GSL = 16
NG = ECH // GSL
EPT = ECH * CH
E_PAD = EPT * NT * NCORE
NPT = NPAD // NT
ROWS = 1024

_mesh = plsc.VectorSubcoreMesh(core_axis_name="c", subcore_axis_name="s")


@functools.partial(
    pl.kernel,
    out_type=[
        jax.ShapeDtypeStruct((NCORE * 2 * NPAD,), jnp.float32),
        jax.ShapeDtypeStruct((E_PAD // CH, CH), jnp.float32),
        jax.ShapeDtypeStruct((E_PAD // CH, CH), jnp.float32),
    ],
    mesh=_mesh,
    scratch_types=[
        pltpu.VMEM((ECH, CH), jnp.int32),
        pltpu.VMEM((ECH, CH), jnp.float32),
        pltpu.VMEM((ECH, CH), jnp.int32),
        pltpu.VMEM((ECH, CH), jnp.int32),
        pltpu.VMEM((ECH, CH), jnp.float32),
        pltpu.VMEM((ECH, CH), jnp.float32),
        pltpu.VMEM((2 * NPAD // NT,), jnp.float32),
        pltpu.VMEM_SHARED((2 * NPAD,), jnp.float32),
    ],
)
def _sc_deg(dst_hbm, ew_hbm, et_hbm, deg_hbm, e0_hbm, e1_hbm,
            dstv, ewv, etv, dstadjv, ewm0v, ewm1v, zv, sh_deg):
    c = lax.axis_index("c")
    s = lax.axis_index("s")
    w = c * NT + s
    npt2 = 2 * NPAD // NT
    pltpu.sync_copy(dst_hbm.at[pl.ds(w * ECH, ECH)], dstv)
    pltpu.sync_copy(ew_hbm.at[pl.ds(w * ECH, ECH)], ewv)
    pltpu.sync_copy(et_hbm.at[pl.ds(w * ECH, ECH)], etv)
    for k in range(npt2 // 16):
        zv[pl.ds(k * 16, 16)] = jnp.zeros((16,), jnp.float32)
    pltpu.sync_copy(zv, sh_deg.at[pl.ds(s * npt2, npt2)])

    def mask_body(j, carry):
        for k in range(CH // 16):
            sl = pl.ds(k * 16, 16)
            et16 = etv[j, sl]
            dstadjv[j, sl] = dstv[j, sl] + et16 * NPAD
            e0v = jnp.where(et16 == 0, ewv[j, sl],
                            jnp.zeros((16,), jnp.float32))
            ewm0v[j, sl] = e0v
            ewm1v[j, sl] = ewv[j, sl] - e0v
        return carry
    lax.fori_loop(0, ECH, mask_body, 0)
    pltpu.sync_copy(ewm0v, e0_hbm.at[pl.ds(w * ECH, ECH)])
    pltpu.sync_copy(ewm1v, e1_hbm.at[pl.ds(w * ECH, ECH)])
    plsc.subcore_barrier()

    def add_body(j, carry):
        pltpu.sync_copy(ewv.at[j], sh_deg.at[dstadjv.at[j]], add=True)
        return carry
    lax.fori_loop(0, ECH, add_body, 0)
    plsc.subcore_barrier()
    pltpu.sync_copy(sh_deg.at[pl.ds(s * npt2, npt2)],
                    deg_hbm.at[pl.ds(c * 2 * NPAD + s * npt2, npt2)])


@functools.partial(
    pl.kernel,
    out_type=jax.ShapeDtypeStruct((NCORE * NPAD, H2), jnp.float32),
    mesh=_mesh,
    scratch_types=[
        pltpu.VMEM((2, GSL, CH), jnp.int32),
        pltpu.VMEM((2, GSL, CH), jnp.int32),
        pltpu.VMEM((2, GSL, CH), jnp.float32),
        pltpu.VMEM((2, GSL, CH), jnp.float32),
        pltpu.VMEM((4, CH, H2), jnp.float32),
        pltpu.VMEM_SHARED((NPAD, H2), jnp.float32),
        pltpu.SemaphoreType.DMA((4,)),
        pltpu.SemaphoreType.DMA((4,)),
        pltpu.SemaphoreType.DMA((2,)),
    ],
)
def _sc_agg(src_hbm, dst_hbm, ew_hbm, e0_hbm, y_hbm, z_hbm,
            src_st, dst_st, ew_st, e0_st, rows, sh_z,
            g_sem, s_sem, st_sem):
    c = lax.axis_index("c")
    s = lax.axis_index("s")
    w = c * NT + s
    tb = w * ECH

    def zrow(r, carry):
        for kk in range(H2 // 16):
            rows[0, r, pl.ds(kk * 16, 16)] = jnp.zeros((16,), jnp.float32)
        return carry
    lax.fori_loop(0, CH, zrow, 0)
    for k in range(NPT // CH):
        pltpu.sync_copy(rows.at[0], sh_z.at[pl.ds(s * NPT + k * CH, CH)])
    plsc.subcore_barrier()

    def issue_stage(g, p):
        b0 = pl.multiple_of(tb + g * GSL, 8)
        pltpu.async_copy(src_hbm.at[pl.ds(b0, GSL)], src_st.at[p],
                         st_sem.at[p])
        pltpu.async_copy(dst_hbm.at[pl.ds(b0, GSL)], dst_st.at[p],
                         st_sem.at[p])
        pltpu.async_copy(ew_hbm.at[pl.ds(b0, GSL)], ew_st.at[p],
                         st_sem.at[p])
        pltpu.async_copy(e0_hbm.at[pl.ds(b0, GSL)], e0_st.at[p],
                         st_sem.at[p])

    def drain_stage(p):
        pltpu.make_async_copy(src_hbm.at[pl.ds(0, GSL)], src_st.at[p],
                              st_sem.at[p]).wait()
        pltpu.make_async_copy(dst_hbm.at[pl.ds(0, GSL)], dst_st.at[p],
                              st_sem.at[p]).wait()
        pltpu.make_async_copy(ew_hbm.at[pl.ds(0, GSL)], ew_st.at[p],
                              st_sem.at[p]).wait()
        pltpu.make_async_copy(e0_hbm.at[pl.ds(0, GSL)], e0_st.at[p],
                              st_sem.at[p]).wait()

    def drain_scatter(bb):
        pltpu.make_async_copy(rows.at[bb], sh_z.at[pl.ds(0, CH)],
                              s_sem.at[bb]).wait()

    issue_stage(0, 0)
    drain_stage(0)
    issue_stage(1, 1)
    pltpu.async_copy(y_hbm.at[src_st.at[0, 0]], rows.at[0], g_sem.at[0])
    pltpu.async_copy(y_hbm.at[src_st.at[0, 1]], rows.at[1], g_sem.at[1])

    def slot(j, carry):
        jm = lax.rem(j, GSL)
        g = lax.div(j, GSL)
        p = lax.rem(g, 2)
        b = lax.rem(j, 4)

        @pl.when(jnp.logical_and(jm == 2,
                                 jnp.logical_and(j >= GSL,
                                                 j < (NG - 1) * GSL)))
        def _():
            issue_stage(g + 1, lax.rem(g + 1, 2))

        @pl.when(jnp.logical_and(jm == GSL - 2, j < (NG - 1) * GSL))
        def _():
            drain_stage(lax.rem(g + 1, 2))

        jj = j + 2
        bb = lax.rem(jj, 4)
        pj = lax.rem(lax.div(jj, GSL), 2)
        jjm = lax.rem(jj, GSL)

        @pl.when(jj >= 4)
        def _():
            drain_scatter(bb)

        @pl.when(jj < ECH)
        def _():
            pltpu.async_copy(y_hbm.at[src_st.at[pj, jjm]], rows.at[bb],
                             g_sem.at[bb])

        pltpu.make_async_copy(y_hbm.at[src_st.at[0, 0]], rows.at[b],
                              g_sem.at[b]).wait()

        def srow(gg, carry2):
            ew16 = ew_st[p, jm, pl.ds(gg * 16, 16)]
            e016 = e0_st[p, jm, pl.ds(gg * 16, 16)]
            for l in range(16):
                r = gg * 16 + l
                sp0 = jnp.full((16,), e016[l], jnp.float32)
                sp1 = jnp.full((16,), ew16[l], jnp.float32)
                for kk in range(H // 16):
                    rows[b, r, pl.ds(kk * 16, 16)] = (
                        rows[b, r, pl.ds(kk * 16, 16)] * sp0)
                for kk in range(H // 16):
                    rows[b, r, pl.ds(H + kk * 16, 16)] = (
                        rows[b, r, pl.ds(H + kk * 16, 16)] * sp1)
            return carry2
        lax.fori_loop(0, CH // 16, srow, 0)

        pltpu.async_copy(rows.at[b], sh_z.at[dst_st.at[p, jm]],
                         s_sem.at[b], add=True)
        return carry
    lax.fori_loop(0, ECH, slot, 0)

    drain_scatter(2)
    drain_scatter(3)
    plsc.subcore_barrier()
    pltpu.sync_copy(sh_z.at[pl.ds(s * NPT, NPT)],
                    z_hbm.at[pl.ds(c * NPAD + s * NPT, NPT)])


def _tc_call(body, grid, in_specs, out_specs, out_shape):
    return pl.pallas_call(
        body, grid=grid, in_specs=in_specs, out_specs=out_specs,
        out_shape=out_shape)


def _row_spec(d):
    return pl.BlockSpec((ROWS, d), lambda i: (i, 0))


def _row2_spec(d):
    return pl.BlockSpec((NCORE, ROWS, d), lambda i: (0, i, 0))


def _full_spec(shape):
    return pl.BlockSpec(shape, lambda i: (0,) * len(shape))


def _tc_h_body(x_ref, w_ref, b_ref, o_ref):
    o_ref[...] = jax.nn.relu(
        jnp.dot(x_ref[...], w_ref[...], preferred_element_type=jnp.float32)
        + b_ref[...])


def _tc_b_body(h_ref, deg_ref, w0_ref, w1_ref, y_ref, dis_ref):
    deg = deg_ref[0] + deg_ref[1]
    dis = 1.0 / jnp.sqrt(deg + 1.0)
    h = h_ref[...]
    y0 = dis[0][:, None] * jnp.dot(h, w0_ref[...],
                                   preferred_element_type=jnp.float32)
    y1 = dis[1][:, None] * jnp.dot(h, w1_ref[...],
                                   preferred_element_type=jnp.float32)
    y_ref[:, 0:H] = y0
    y_ref[:, H:H2] = y1
    dis_ref[...] = dis


def _tc_c_body(z_ref, y_ref, dis_ref, b_ref, g_ref, be_ref, w_ref,
               o_ref, y2_ref):
    inv = 1.0 / jnp.sqrt(1.0 + EPS)
    zz = z_ref[0] + z_ref[1] + y_ref[...]
    y2 = []
    for t in range(2):
        dis = dis_ref[t][:, None]
        agg = dis * zz[:, t * H:(t + 1) * H] + b_ref[t]
        o = jax.nn.relu(g_ref[t] * (agg * inv) + be_ref[t])
        o_ref[t] = o
        y2t = dis * jnp.dot(o, w_ref[t], preferred_element_type=jnp.float32)
        y2_ref[:, t * H:(t + 1) * H] = y2t


def _tc_d_body(z_ref, y2_ref, o_ref, dis_ref, b_ref, g_ref, be_ref, x_ref):
    inv = 1.0 / jnp.sqrt(1.0 + EPS)
    zz = z_ref[0] + z_ref[1] + y2_ref[...]
    for t in range(2):
        dis = dis_ref[t][:, None]
        agg = dis * zz[:, t * H:(t + 1) * H] + b_ref[t]
        x_ref[t] = g_ref[t] * (agg * inv) + be_ref[t] + o_ref[t]


def _tc_s_body(x_ref, seed_ref, wcd_ref, bp1_ref, sb_ref):
    sd = seed_ref[0]
    x0r = x_ref[0, pl.ds(sd, 1), :]
    x1r = x_ref[1, pl.ds(sd, 1), :]
    sb_ref[...] = (
        jnp.dot(x0r, wcd_ref[0], preferred_element_type=jnp.float32)
        + jnp.dot(x1r, wcd_ref[1], preferred_element_type=jnp.float32)
        + bp1_ref[...])


def _tc_e_body(x_ref, sb_ref, wab_ref, w2_ref, b2_ref, w3_ref, b3_ref, o_ref):
    o = jax.nn.relu(
        jnp.dot(x_ref[0], wab_ref[0], preferred_element_type=jnp.float32)
        + jnp.dot(x_ref[1], wab_ref[1], preferred_element_type=jnp.float32)
        + sb_ref[...])
    o = jax.nn.relu(jnp.dot(o, w2_ref[...], preferred_element_type=jnp.float32)
                    + b2_ref[...])
    o_ref[...] = jnp.dot(o, w3_ref[...], preferred_element_type=jnp.float32) \
        + b3_ref[...]


def kernel(x, edge_index, edge_type, edge_weight, seed_node_id, W_ft, b_ft, W00, b00, g00, be00, W01, b01, g01, be01, W10, b10, g10, be10, W11, b11, g11, be11, Wp1, bp1, Wp2, bp2, Wp3, bp3):
    f32 = jnp.float32
    grid = (NPAD // ROWS,)

    xp = jnp.pad(x, ((0, NPAD - N), (0, 0)))
    nrow = E_PAD // CH
    src = jnp.pad(edge_index[0], (0, E_PAD - E)).reshape(nrow, CH)
    dst = jnp.pad(edge_index[1], (0, E_PAD - E)).reshape(nrow, CH)
    ew = jnp.pad(edge_weight, (0, E_PAD - E)).reshape(nrow, CH)
    et = jnp.pad(edge_type, (0, E_PAD - E)).reshape(nrow, CH)
    seed = jnp.asarray(seed_node_id, jnp.int32).reshape(1)
    bL0 = jnp.stack([b00, b10]); gL0 = jnp.stack([g00, g10])
    beL0 = jnp.stack([be00, be10])
    WL1 = jnp.stack([W01, W11])
    bL1 = jnp.stack([b01, b11]); gL1 = jnp.stack([g01, g11])
    beL1 = jnp.stack([be01, be11])
    Wp1ab = jnp.stack([Wp1[0:H], Wp1[H:2 * H]])
    Wp1cd = jnp.stack([Wp1[2 * H:3 * H], Wp1[3 * H:4 * H]])

    deg, ewm0, ewm1 = _sc_deg(dst, ew, et)
    deg4 = deg.reshape(NCORE, 2, NPAD)

    h = _tc_call(
        _tc_h_body, grid,
        [_row_spec(128), _full_spec((128, H)), _full_spec((H,))],
        _row_spec(H), jax.ShapeDtypeStruct((NPAD, H), f32))(xp, W_ft, b_ft)

    y, dis = _tc_call(
        _tc_b_body, grid,
        [_row_spec(H), pl.BlockSpec((NCORE, 2, ROWS), lambda i: (0, 0, i)),
         _full_spec((H, H)), _full_spec((H, H))],
        [_row_spec(H2), pl.BlockSpec((2, ROWS), lambda i: (0, i))],
        [jax.ShapeDtypeStruct((NPAD, H2), f32),
         jax.ShapeDtypeStruct((2, NPAD), f32)])(h, deg4, W00, W10)

    z = _sc_agg(src, dst, ewm1, ewm0, y)
    z = z.reshape(NCORE, NPAD, H2)

    o, y2 = _tc_call(
        _tc_c_body, grid,
        [_row2_spec(H2), _row_spec(H2),
         pl.BlockSpec((2, ROWS), lambda i: (0, i)),
         _full_spec((2, H)), _full_spec((2, H)),
         _full_spec((2, H)), _full_spec((2, H, H))],
        [_row2_spec(H), _row_spec(H2)],
        [jax.ShapeDtypeStruct((NCORE, NPAD, H), f32),
         jax.ShapeDtypeStruct((NPAD, H2), f32)])(
             z, y, dis, bL0, gL0, beL0, WL1)

    z2 = _sc_agg(src, dst, ewm1, ewm0, y2)
    z2 = z2.reshape(NCORE, NPAD, H2)

    x01 = _tc_call(
        _tc_d_body, grid,
        [_row2_spec(H2), _row_spec(H2), _row2_spec(H),
         pl.BlockSpec((2, ROWS), lambda i: (0, i)),
         _full_spec((2, H)), _full_spec((2, H)), _full_spec((2, H))],
        _row2_spec(H),
        jax.ShapeDtypeStruct((NCORE, NPAD, H), f32))(
            z2, y2, o, dis, bL1, gL1, beL1)

    sb = pl.pallas_call(
        _tc_s_body,
        grid=(1,),
        in_specs=[_full_spec((NCORE, NPAD, H)),
                  pl.BlockSpec(memory_space=pltpu.SMEM),
                  _full_spec((NCORE, H, H)), _full_spec((H,))],
        out_specs=_full_spec((1, H)),
        out_shape=jax.ShapeDtypeStruct((1, H), f32))(x01, seed, Wp1cd, bp1)

    res = _tc_call(
        _tc_e_body, grid,
        [_row2_spec(H), _full_spec((1, H)), _full_spec((NCORE, H, H)),
         _full_spec((H, H)), _full_spec((H,)), _full_spec((H, 1)),
         _full_spec((1,))],
        _row_spec(1),
        jax.ShapeDtypeStruct((NPAD, 1), f32))(
            x01, sb, Wp1ab, Wp2, bp2, Wp3, bp3)

    return res[:N, 0]
